# baseline (speedup 1.0000x reference)
import functools

import jax
import jax.numpy as jnp
from jax import lax
from jax.experimental import pallas as pl
from jax.experimental.pallas import tpu as pltpu
from jax.experimental.pallas import tpu_sc as plsc

N_ROWS = 50000
N_PAD = 50048
D_IN = 128
HID = 128
C_OUT = 10
F_TRIM = 100
CHUNK = 2944
N_CHUNKS = N_PAD // CHUNK
NW = 32
COLS_PER_W = HID // NW
UNROLL = 5
STEP = 16 * UNROLL
FULL_ITERS = N_ROWS // STEP
CBUF = 50096
HB = 2048


def _mmT_kernel(w1_ref, x_ref, b1_ref, ht_ref):
    ht_ref[...] = jnp.maximum(
        lax.dot_general(
            w1_ref[...], x_ref[...], (((0,), (1,)), ((), ())),
            preferred_element_type=jnp.float32,
        )
        + b1_ref[...],
        0.0,
    )


def _matmul_T(x, W1, b1c):
    return pl.pallas_call(
        _mmT_kernel,
        grid=(N_CHUNKS,),
        in_specs=[
            pl.BlockSpec((D_IN, HID), lambda i: (0, 0)),
            pl.BlockSpec((CHUNK, D_IN), lambda i: (i, 0)),
            pl.BlockSpec((HID, 1), lambda i: (0, 0)),
        ],
        out_specs=pl.BlockSpec((HID, CHUNK), lambda i: (0, i)),
        out_shape=jax.ShapeDtypeStruct((HID, N_PAD), jnp.float32),
    )(W1, x, b1c)


def _splat(s):
    return lax.broadcast_in_dim(s, (16,), ())


_SC_MESH = plsc.VectorSubcoreMesh(core_axis_name="c", subcore_axis_name="s")


@functools.partial(
    pl.kernel,
    mesh=_SC_MESH,
    compiler_params=pltpu.CompilerParams(needs_layout_passes=False),
    out_type=jax.ShapeDtypeStruct((NW, 16), jnp.float32),
    scratch_types=[
        pltpu.VMEM((N_PAD,), jnp.float32),
        pltpu.VMEM((CBUF,), jnp.float32),
        pltpu.VMEM((HB,), jnp.float32),
        pltpu.VMEM((16,), jnp.float32),
        pltpu.SemaphoreType.DMA,
    ],
)
def _sc_select(ht_hbm, out_hbm, col_v, cbuf, hist, res_v, dma_sem):
    wid = lax.axis_index("s") * 2 + lax.axis_index("c")
    ones = jnp.ones((16,), jnp.float32)
    zeros16 = jnp.zeros((16,), jnp.float32)
    izeros16 = jnp.zeros((16,), jnp.int32)
    lane = lax.iota(jnp.int32, 16)
    f_v = jnp.full((16,), float(F_TRIM), jnp.float32)
    n_v = jnp.full((16,), float(N_ROWS), jnp.float32)

    def zero_hist(nchunks):
        def zb(i, c):
            hist[pl.ds(i * 16, 16)] = zeros16
            return c

        lax.fori_loop(0, nchunks, zb, 0)

    def add_zeros_to_bucket0(zb):
        h0 = hist[pl.ds(0, 16)]
        hist[pl.ds(0, 16)] = h0 + jnp.where(lane == 0, zb, zeros16)

    def pick(nchunks, k_rem):
        def body(i, acc):
            nlt, cadd, run = acc
            h = hist[pl.ds(i * 16, 16)]
            cs = plsc.cumsum(h) + run
            lt = cs < k_rem
            nlt = nlt + jnp.where(lt, 1.0, 0.0)
            cadd = cadd + jnp.where(lt, h, 0.0)
            run = run + _splat(jnp.sum(h))
            return nlt, cadd, run

        nlt, cadd, _ = lax.fori_loop(
            0, nchunks, body, (zeros16, zeros16, zeros16)
        )
        return _splat(jnp.sum(nlt)).astype(jnp.int32), _splat(jnp.sum(cadd))

    def nvecs(n_splat):
        return lax.shift_right_logical(jnp.max(n_splat) + 15, 4)

    def region_sum_lt(start_s, nk, t):
        def body(i, acc):
            v = cbuf[pl.ds(start_s + i * 16, 16)]
            valid = (i * 16 + lane) < nk
            return acc + jnp.where(valid & (v < t), v, 0.0)

        return _splat(jnp.sum(lax.fori_loop(0, nvecs(nk), body, zeros16)))

    res = zeros16
    col0 = wid * COLS_PER_W
    dma = pltpu.async_copy(ht_hbm.at[col0], col_v, dma_sem)
    for j in range(COLS_PER_W):
        dma.wait()

        zero_hist(HB // 16)

        def scan_a(i, acc):
            zc, tot = acc
            for u in range(UNROLL):
                v = col_v[pl.ds(i * STEP + u * 16, 16)]
                nz = v > 0.0
                bits = lax.bitcast_convert_type(v, jnp.int32)
                f0 = lax.shift_right_logical(bits, 21)
                plsc.addupdate_scatter(hist, [f0], ones, mask=nz)
                zc = zc + jnp.where(nz, 0.0, 1.0)
                tot = tot + v
            return zc, tot

        zc, totv = lax.fori_loop(0, FULL_ITERS, scan_a, (zeros16, zeros16))
        z = _splat(jnp.sum(zc))
        total = _splat(jnp.sum(totv))

        add_zeros_to_bucket0(z)
        k_lo0 = f_v
        k_hi0 = jnp.full((16,), float(N_ROWS - F_TRIM + 1), jnp.float32)

        def pick2_body(i, acc):
            nl, cl, nh, ch, run = acc
            h = hist[pl.ds(i * 16, 16)]
            cs = plsc.cumsum(h) + run
            lt = cs < k_lo0
            nl = nl + jnp.where(lt, 1.0, 0.0)
            cl = cl + jnp.where(lt, h, 0.0)
            lt = cs < k_hi0
            nh = nh + jnp.where(lt, 1.0, 0.0)
            ch = ch + jnp.where(lt, h, 0.0)
            run = run + _splat(jnp.sum(h))
            return nl, cl, nh, ch, run

        nl, cl, nh, ch, _ = lax.fori_loop(
            0, HB // 16, pick2_body,
            (zeros16, zeros16, zeros16, zeros16, zeros16),
        )
        b0_lo = _splat(jnp.sum(nl)).astype(jnp.int32)
        k_lo = k_lo0 - _splat(jnp.sum(cl))
        b0_hi = _splat(jnp.sum(nh)).astype(jnp.int32)
        k_hi = k_hi0 - _splat(jnp.sum(ch))

        neq = b0_lo != b0_hi

        def compact_both(i, acc):
            for u in range(UNROLL):
                v = col_v[pl.ds(i * STEP + u * 16, 16)]
                nz = v > 0.0
                bits = lax.bitcast_convert_type(v, jnp.int32)
                f0 = lax.shift_right_logical(bits, 21)
                rare = nz & ((f0 <= b0_lo) | (f0 >= b0_hi))

                def slow(carry):
                    w_lo, w_hi, sb_lo, sb_ge = carry
                    sb_lo = sb_lo + jnp.where(f0 < b0_lo, v, 0.0)
                    sb_ge = sb_ge + jnp.where(f0 >= b0_hi, v, 0.0)
                    m_lo = (f0 == b0_lo) & nz
                    c_lo = plsc.cumsum(m_lo.astype(jnp.int32))
                    idx_lo = jnp.maximum(w_lo + c_lo - 1, izeros16)
                    plsc.store_scatter(cbuf, [idx_lo], v, mask=m_lo)
                    w_lo = w_lo + plsc.all_reduce_population_count(m_lo)
                    m_hi = (f0 == b0_hi) & nz & neq
                    c_hi = plsc.cumsum(m_hi.astype(jnp.int32))
                    idx_hi = jnp.clip(CBUF - (w_hi + c_hi), 0, CBUF - 1)
                    plsc.store_scatter(cbuf, [idx_hi], v, mask=m_hi)
                    w_hi = w_hi + plsc.all_reduce_population_count(m_hi)
                    return w_lo, w_hi, sb_lo, sb_ge

                pred = jnp.max(plsc.all_reduce_population_count(rare)) > 0
                acc = lax.cond(pred, slow, lambda c: c, acc)
            return acc

        w_lo, w_hi, sbv_lo, sbv_ge = lax.fori_loop(
            0, FULL_ITERS, compact_both, (izeros16, izeros16, zeros16, zeros16)
        )
        s_below_lo = _splat(jnp.sum(sbv_lo))
        s_below_hi = total - _splat(jnp.sum(sbv_ge))

        if j < COLS_PER_W - 1:
            dma = pltpu.async_copy(ht_hbm.at[col0 + j + 1], col_v, dma_sem)
        eq_s = jnp.max(b0_lo) == jnp.max(b0_hi)
        start_lo = 0
        start_hi = jnp.where(eq_s, 0, CBUF - jnp.max(w_hi))
        nk_lo = w_lo
        nk_hi = jnp.where(neq, w_hi, w_lo)

        def refine(k_rem, b0, start_s, nk):
            zb = jnp.where(b0 == 0, z, zeros16)
            pfx = b0
            nv = nvecs(nk)
            zero_hist(HB // 16)

            def h1(i, c):
                v = cbuf[pl.ds(start_s + i * 16, 16)]
                valid = (i * 16 + lane) < nk
                bits = lax.bitcast_convert_type(v, jnp.int32)
                f1 = jnp.bitwise_and(lax.shift_right_logical(bits, 10), 2047)
                plsc.addupdate_scatter(hist, [f1], ones, mask=valid)
                return c

            lax.fori_loop(0, nv, h1, 0)
            add_zeros_to_bucket0(zb)
            b1, ca1 = pick(HB // 16, k_rem)
            k_rem = k_rem - ca1
            zb = jnp.where(b1 == 0, zb, zeros16)
            pfx = lax.shift_left(pfx, 11) + b1
            zero_hist(64)

            def h2(i, c):
                v = cbuf[pl.ds(start_s + i * 16, 16)]
                valid = (i * 16 + lane) < nk
                bits = lax.bitcast_convert_type(v, jnp.int32)
                f1 = jnp.bitwise_and(lax.shift_right_logical(bits, 10), 2047)
                m = (f1 == b1) & valid
                f2 = jnp.bitwise_and(bits, 1023)
                plsc.addupdate_scatter(hist, [f2], ones, mask=m)
                return c

            lax.fori_loop(0, nv, h2, 0)
            add_zeros_to_bucket0(zb)
            b2, ca2 = pick(64, k_rem)
            k_rem = k_rem - ca2
            pfx = lax.shift_left(pfx, 10) + b2
            return lax.bitcast_convert_type(pfx, jnp.float32), k_rem

        t_lo, krem_lo = refine(k_lo, b0_lo, start_lo, nk_lo)
        t_hi, krem_hi = refine(k_hi, b0_hi, start_hi, nk_hi)
        c_lt_lo = k_lo0 - krem_lo
        c_lt_hi = k_hi0 - krem_hi

        s_lt_lo = s_below_lo + region_sum_lt(start_lo, nk_lo, t_lo)
        s_lt_hi = s_below_hi + region_sum_lt(start_hi, nk_hi, t_hi)

        bot = s_lt_lo + (f_v - c_lt_lo) * t_lo
        top_rm = (total - s_lt_hi) - (n_v - c_lt_hi - f_v) * t_hi
        hbar = (total - bot - top_rm) * (1.0 / (N_ROWS - 2 * F_TRIM))
        res = jnp.where(lane == j, hbar, res)

    res_v[...] = res
    pltpu.sync_copy(res_v, out_hbm.at[wid])


def _dec_kernel(h_ref, w2_ref, b2_ref, o_ref):
    o_ref[...] = (
        jnp.dot(h_ref[...], w2_ref[...], preferred_element_type=jnp.float32)
        + b2_ref[...]
    )


def _decode(hbar, W2p, b2p):
    return pl.pallas_call(
        _dec_kernel,
        out_shape=jax.ShapeDtypeStruct((1, 128), jnp.float32),
    )(hbar, W2p, b2p)


def kernel(x, W1, b1, W2, b2):
    xp = jnp.zeros((N_PAD, D_IN), jnp.float32).at[:N_ROWS].set(x)
    ht = _matmul_T(xp, W1, b1.reshape(HID, 1))
    sel = _sc_select(ht)
    hbar = sel[:, :COLS_PER_W].reshape(1, HID)
    W2p = jnp.zeros((HID, 128), jnp.float32).at[:, :C_OUT].set(W2)
    b2p = jnp.zeros((1, 128), jnp.float32).at[0, :C_OUT].set(b2)
    return _decode(hbar, W2p, b2p)[0, :C_OUT]


# speedup vs baseline: 1.5771x; 1.5771x over previous
import functools

import jax
import jax.numpy as jnp
from jax import lax
from jax.experimental import pallas as pl
from jax.experimental.pallas import tpu as pltpu
from jax.experimental.pallas import tpu_sc as plsc

N_ROWS = 50000
N_PAD = 50048
D_IN = 128
HID = 128
C_OUT = 10
F_TRIM = 100
CHUNK = 2944
N_CHUNKS = N_PAD // CHUNK
NW = 32
COLS_PER_W = HID // NW
UNROLL = 5
STEP = 16 * UNROLL
FULL_ITERS = N_ROWS // STEP
CBUF = 50096
HB = 2048


def _mmT_kernel(w1_ref, x_ref, b1_ref, ht_ref):
    ht_ref[...] = jnp.maximum(
        lax.dot_general(
            w1_ref[...], x_ref[...], (((0,), (1,)), ((), ())),
            preferred_element_type=jnp.float32,
        )
        + b1_ref[...],
        0.0,
    )


def _matmul_T(x, W1, b1c):
    return pl.pallas_call(
        _mmT_kernel,
        grid=(N_CHUNKS,),
        in_specs=[
            pl.BlockSpec((D_IN, HID), lambda i: (0, 0)),
            pl.BlockSpec((CHUNK, D_IN), lambda i: (i, 0)),
            pl.BlockSpec((HID, 1), lambda i: (0, 0)),
        ],
        out_specs=pl.BlockSpec((HID, CHUNK), lambda i: (0, i)),
        out_shape=jax.ShapeDtypeStruct((HID, N_PAD), jnp.float32),
    )(W1, x, b1c)


def _splat(s):
    return lax.broadcast_in_dim(s, (16,), ())


_SC_MESH = plsc.VectorSubcoreMesh(core_axis_name="c", subcore_axis_name="s")


@functools.partial(
    pl.kernel,
    mesh=_SC_MESH,
    compiler_params=pltpu.CompilerParams(needs_layout_passes=False),
    out_type=jax.ShapeDtypeStruct((NW, 16), jnp.float32),
    scratch_types=[
        pltpu.VMEM((N_PAD,), jnp.float32),
        pltpu.VMEM((CBUF,), jnp.float32),
        pltpu.VMEM((HB,), jnp.float32),
        pltpu.VMEM((16,), jnp.float32),
        pltpu.SemaphoreType.DMA,
    ],
)
def _sc_select(ht_hbm, out_hbm, col_v, cbuf, hist, res_v, dma_sem):
    wid = lax.axis_index("s") * 2 + lax.axis_index("c")
    ones = jnp.ones((16,), jnp.float32)
    zeros16 = jnp.zeros((16,), jnp.float32)
    izeros16 = jnp.zeros((16,), jnp.int32)
    lane = lax.iota(jnp.int32, 16)
    f_v = jnp.full((16,), float(F_TRIM), jnp.float32)
    n_v = jnp.full((16,), float(N_ROWS), jnp.float32)

    def zero_hist(nchunks):
        def zb(i, c):
            hist[pl.ds(i * 16, 16)] = zeros16
            return c

        lax.fori_loop(0, nchunks, zb, 0)

    def add_zeros_to_bucket0(zb):
        h0 = hist[pl.ds(0, 16)]
        hist[pl.ds(0, 16)] = h0 + jnp.where(lane == 0, zb, zeros16)

    def pick(nchunks, k_rem):
        def body(i, acc):
            nlt, cadd, run = acc
            h = hist[pl.ds(i * 16, 16)]
            cs = plsc.cumsum(h) + run
            lt = cs < k_rem
            nlt = nlt + jnp.where(lt, 1.0, 0.0)
            cadd = cadd + jnp.where(lt, h, 0.0)
            run = run + _splat(jnp.sum(h))
            return nlt, cadd, run

        nlt, cadd, _ = lax.fori_loop(
            0, nchunks, body, (zeros16, zeros16, zeros16)
        )
        return _splat(jnp.sum(nlt)).astype(jnp.int32), _splat(jnp.sum(cadd))

    def nvecs(n_splat):
        return lax.shift_right_logical(jnp.max(n_splat) + 15, 4)

    def region_sum_lt(start_s, nk, t):
        def body(i, acc):
            v = cbuf[pl.ds(start_s + i * 16, 16)]
            valid = (i * 16 + lane) < nk
            return acc + jnp.where(valid & (v < t), v, 0.0)

        return _splat(jnp.sum(lax.fori_loop(0, nvecs(nk), body, zeros16)))

    res = zeros16
    col0 = wid * COLS_PER_W
    dma = pltpu.async_copy(ht_hbm.at[col0], col_v, dma_sem)
    for j in range(COLS_PER_W):
        dma.wait()

        zero_hist(HB // 16)

        def scan_a(i, tot):
            for u in range(UNROLL):
                v = col_v[pl.ds(i * STEP + u * 16, 16)]
                nz = v > 0.0
                bits = lax.bitcast_convert_type(v, jnp.int32)
                f0 = lax.shift_right_logical(bits, 21)
                plsc.addupdate_scatter(hist, [f0], ones, mask=nz)
                tot = tot + v
            return tot

        totv = lax.fori_loop(0, FULL_ITERS, scan_a, zeros16)
        total = _splat(jnp.sum(totv))

        def csum_body(i, acc):
            return acc + hist[pl.ds(i * 16, 16)]

        cnz = _splat(jnp.sum(lax.fori_loop(0, HB // 16, csum_body, zeros16)))
        z = n_v - cnz

        k_lo0 = f_v
        k_hi0 = jnp.full((16,), float(N_ROWS - F_TRIM + 1), jnp.float32)
        kz_lo = k_lo0 - z
        kz_hi = k_hi0 - z

        def pick2_body(i, acc):
            nl, cl, nh, ch, run = acc
            h = hist[pl.ds(i * 16, 16)]
            cs = plsc.cumsum(h) + run
            lt = cs < kz_lo
            nl = nl + jnp.where(lt, 1.0, 0.0)
            cl = cl + jnp.where(lt, h, 0.0)
            lt = cs < kz_hi
            nh = nh + jnp.where(lt, 1.0, 0.0)
            ch = ch + jnp.where(lt, h, 0.0)
            run = run + _splat(jnp.sum(h))
            return nl, cl, nh, ch, run

        nl, cl, nh, ch, _ = lax.fori_loop(
            0, HB // 16, pick2_body,
            (zeros16, zeros16, zeros16, zeros16, zeros16),
        )
        b0_lo = _splat(jnp.sum(nl)).astype(jnp.int32)
        k_lo = k_lo0 - _splat(jnp.sum(cl)) - jnp.where(b0_lo > 0, z, zeros16)
        b0_hi = _splat(jnp.sum(nh)).astype(jnp.int32)
        k_hi = k_hi0 - _splat(jnp.sum(ch)) - jnp.where(b0_hi > 0, z, zeros16)

        neq = b0_lo != b0_hi

        def compact_both(i, acc):
            w_lo, w_hi, sb_lo, sb_hi = acc
            for u in range(UNROLL):
                v = col_v[pl.ds(i * STEP + u * 16, 16)]
                nz = v > 0.0
                bits = lax.bitcast_convert_type(v, jnp.int32)
                f0 = lax.shift_right_logical(bits, 21)
                sb_lo = sb_lo + jnp.where(f0 < b0_lo, v, 0.0)
                sb_hi = sb_hi + jnp.where(f0 < b0_hi, v, 0.0)
                m_lo = (f0 == b0_lo) & nz
                c_lo = plsc.cumsum(m_lo.astype(jnp.int32))
                idx_lo = jnp.maximum(w_lo + c_lo - 1, izeros16)
                plsc.store_scatter(cbuf, [idx_lo], v, mask=m_lo)
                w_lo = w_lo + plsc.all_reduce_population_count(m_lo)
                m_hi = (f0 == b0_hi) & nz & neq
                c_hi = plsc.cumsum(m_hi.astype(jnp.int32))
                idx_hi = jnp.clip(CBUF - (w_hi + c_hi), 0, CBUF - 1)
                plsc.store_scatter(cbuf, [idx_hi], v, mask=m_hi)
                w_hi = w_hi + plsc.all_reduce_population_count(m_hi)
            return w_lo, w_hi, sb_lo, sb_hi

        w_lo, w_hi, sbv_lo, sbv_hi = lax.fori_loop(
            0, FULL_ITERS, compact_both, (izeros16, izeros16, zeros16, zeros16)
        )
        s_below_lo = _splat(jnp.sum(sbv_lo))
        s_below_hi = _splat(jnp.sum(sbv_hi))

        if j < COLS_PER_W - 1:
            dma = pltpu.async_copy(ht_hbm.at[col0 + j + 1], col_v, dma_sem)
        eq_s = jnp.max(b0_lo) == jnp.max(b0_hi)
        start_lo = 0
        start_hi = jnp.where(eq_s, 0, CBUF - jnp.max(w_hi))
        nk_lo = w_lo
        nk_hi = jnp.where(neq, w_hi, w_lo)

        def refine(k_rem, b0, start_s, nk):
            zb = jnp.where(b0 == 0, z, zeros16)
            pfx = b0
            nv = nvecs(nk)
            zero_hist(HB // 16)

            def h1(i, c):
                v = cbuf[pl.ds(start_s + i * 16, 16)]
                valid = (i * 16 + lane) < nk
                bits = lax.bitcast_convert_type(v, jnp.int32)
                f1 = jnp.bitwise_and(lax.shift_right_logical(bits, 10), 2047)
                plsc.addupdate_scatter(hist, [f1], ones, mask=valid)
                return c

            lax.fori_loop(0, nv, h1, 0)
            add_zeros_to_bucket0(zb)
            b1, ca1 = pick(HB // 16, k_rem)
            k_rem = k_rem - ca1
            zb = jnp.where(b1 == 0, zb, zeros16)
            pfx = lax.shift_left(pfx, 11) + b1
            zero_hist(64)

            def h2(i, c):
                v = cbuf[pl.ds(start_s + i * 16, 16)]
                valid = (i * 16 + lane) < nk
                bits = lax.bitcast_convert_type(v, jnp.int32)
                f1 = jnp.bitwise_and(lax.shift_right_logical(bits, 10), 2047)
                m = (f1 == b1) & valid
                f2 = jnp.bitwise_and(bits, 1023)
                plsc.addupdate_scatter(hist, [f2], ones, mask=m)
                return c

            lax.fori_loop(0, nv, h2, 0)
            add_zeros_to_bucket0(zb)
            b2, ca2 = pick(64, k_rem)
            k_rem = k_rem - ca2
            pfx = lax.shift_left(pfx, 10) + b2
            return lax.bitcast_convert_type(pfx, jnp.float32), k_rem

        t_lo, krem_lo = refine(k_lo, b0_lo, start_lo, nk_lo)
        t_hi, krem_hi = refine(k_hi, b0_hi, start_hi, nk_hi)
        c_lt_lo = k_lo0 - krem_lo
        c_lt_hi = k_hi0 - krem_hi

        s_lt_lo = s_below_lo + region_sum_lt(start_lo, nk_lo, t_lo)
        s_lt_hi = s_below_hi + region_sum_lt(start_hi, nk_hi, t_hi)

        bot = s_lt_lo + (f_v - c_lt_lo) * t_lo
        top_rm = (total - s_lt_hi) - (n_v - c_lt_hi - f_v) * t_hi
        hbar = (total - bot - top_rm) * (1.0 / (N_ROWS - 2 * F_TRIM))
        res = jnp.where(lane == j, hbar, res)

    res_v[...] = res
    pltpu.sync_copy(res_v, out_hbm.at[wid])


def _dec_kernel(h_ref, w2_ref, b2_ref, o_ref):
    o_ref[...] = (
        jnp.dot(h_ref[...], w2_ref[...], preferred_element_type=jnp.float32)
        + b2_ref[...]
    )


def _decode(hbar, W2p, b2p):
    return pl.pallas_call(
        _dec_kernel,
        out_shape=jax.ShapeDtypeStruct((1, 128), jnp.float32),
    )(hbar, W2p, b2p)


def kernel(x, W1, b1, W2, b2):
    xp = jnp.zeros((N_PAD, D_IN), jnp.float32).at[:N_ROWS].set(x)
    ht = _matmul_T(xp, W1, b1.reshape(HID, 1))
    sel = _sc_select(ht)
    hbar = sel[:, :COLS_PER_W].reshape(1, HID)
    W2p = jnp.zeros((HID, 128), jnp.float32).at[:, :C_OUT].set(W2)
    b2p = jnp.zeros((1, 128), jnp.float32).at[0, :C_OUT].set(b2)
    return _decode(hbar, W2p, b2p)[0, :C_OUT]


# speedup vs baseline: 1.6043x; 1.0172x over previous
import functools

import jax
import jax.numpy as jnp
from jax import lax
from jax.experimental import pallas as pl
from jax.experimental.pallas import tpu as pltpu
from jax.experimental.pallas import tpu_sc as plsc

N_ROWS = 50000
N_PAD = 50048
D_IN = 128
HID = 128
C_OUT = 10
F_TRIM = 100
CHUNK = 2944
N_CHUNKS = N_PAD // CHUNK
NW = 32
COLS_PER_W = HID // NW
UNROLL = 25
STEP = 16 * UNROLL
FULL_ITERS = N_ROWS // STEP
CBUF = 50096
HB = 1024
RB = 128


def _mmT_kernel(w1_ref, x_ref, b1_ref, ht_ref):
    ht_ref[...] = jnp.maximum(
        lax.dot_general(
            w1_ref[...], x_ref[...], (((0,), (1,)), ((), ())),
            preferred_element_type=jnp.float32,
        )
        + b1_ref[...],
        0.0,
    )


def _matmul_T(x, W1, b1c):
    return pl.pallas_call(
        _mmT_kernel,
        grid=(N_CHUNKS,),
        in_specs=[
            pl.BlockSpec((D_IN, HID), lambda i: (0, 0)),
            pl.BlockSpec((CHUNK, D_IN), lambda i: (i, 0)),
            pl.BlockSpec((HID, 1), lambda i: (0, 0)),
        ],
        out_specs=pl.BlockSpec((HID, CHUNK), lambda i: (0, i)),
        out_shape=jax.ShapeDtypeStruct((HID, N_PAD), jnp.float32),
    )(W1, x, b1c)


def _splat(s):
    return lax.broadcast_in_dim(s, (16,), ())


_SC_MESH = plsc.VectorSubcoreMesh(core_axis_name="c", subcore_axis_name="s")


@functools.partial(
    pl.kernel,
    mesh=_SC_MESH,
    compiler_params=pltpu.CompilerParams(needs_layout_passes=False),
    out_type=jax.ShapeDtypeStruct((NW, 16), jnp.float32),
    scratch_types=[
        pltpu.VMEM((N_PAD,), jnp.float32),
        pltpu.VMEM((CBUF,), jnp.float32),
        pltpu.VMEM((HB,), jnp.float32),
        pltpu.VMEM((16,), jnp.float32),
        pltpu.SemaphoreType.DMA,
    ],
)
def _sc_select(ht_hbm, out_hbm, col_v, cbuf, hist, res_v, dma_sem):
    wid = lax.axis_index("s") * 2 + lax.axis_index("c")
    ones = jnp.ones((16,), jnp.float32)
    zeros16 = jnp.zeros((16,), jnp.float32)
    izeros16 = jnp.zeros((16,), jnp.int32)
    lane = lax.iota(jnp.int32, 16)
    f_v = jnp.full((16,), float(F_TRIM), jnp.float32)
    n_v = jnp.full((16,), float(N_ROWS), jnp.float32)

    def zero_hist(nchunks):
        def zb(i, c):
            hist[pl.ds(i * 16, 16)] = zeros16
            return c

        lax.fori_loop(0, nchunks, zb, 0)

    def add_zeros_to_bucket0(zb):
        h0 = hist[pl.ds(0, 16)]
        hist[pl.ds(0, 16)] = h0 + jnp.where(lane == 0, zb, zeros16)

    def pick(nchunks, k_rem):
        def body(i, acc):
            nlt, cadd, run = acc
            h = hist[pl.ds(i * 16, 16)]
            cs = plsc.cumsum(h) + run
            lt = cs < k_rem
            nlt = nlt + jnp.where(lt, 1.0, 0.0)
            cadd = cadd + jnp.where(lt, h, 0.0)
            run = run + _splat(jnp.sum(h))
            return nlt, cadd, run

        nlt, cadd, _ = lax.fori_loop(
            0, nchunks, body, (zeros16, zeros16, zeros16)
        )
        return _splat(jnp.sum(nlt)).astype(jnp.int32), _splat(jnp.sum(cadd))

    def nvecs(n_splat):
        return lax.shift_right_logical(jnp.max(n_splat) + 15, 4)

    def region_sum_lt(start_s, nk, t):
        def body(i, acc):
            v = cbuf[pl.ds(start_s + i * 16, 16)]
            valid = (i * 16 + lane) < nk
            return acc + jnp.where(valid & (v < t), v, 0.0)

        return _splat(jnp.sum(lax.fori_loop(0, nvecs(nk), body, zeros16)))

    res = zeros16
    col0 = wid * COLS_PER_W
    dma = pltpu.async_copy(ht_hbm.at[col0], col_v, dma_sem)
    for j in range(COLS_PER_W):
        dma.wait()

        zero_hist(HB // 16)

        def scan_a(i, tot):
            for u in range(UNROLL):
                v = col_v[pl.ds(i * STEP + u * 16, 16)]
                nz = v > 0.0
                bits = lax.bitcast_convert_type(v, jnp.int32)
                f0 = lax.shift_right_logical(bits, 21)
                plsc.addupdate_scatter(hist, [f0], ones, mask=nz)
                tot = tot + v
            return tot

        totv = lax.fori_loop(0, FULL_ITERS, scan_a, zeros16)
        total = _splat(jnp.sum(totv))

        def csum_body(i, acc):
            return acc + hist[pl.ds(i * 16, 16)]

        cnz = _splat(jnp.sum(lax.fori_loop(0, HB // 16, csum_body, zeros16)))
        z = n_v - cnz

        k_lo0 = f_v
        k_hi0 = jnp.full((16,), float(N_ROWS - F_TRIM + 1), jnp.float32)
        kz_lo = k_lo0 - z
        kz_hi = k_hi0 - z

        def pick2_body(i, acc):
            nl, cl, nh, ch, run = acc
            h = hist[pl.ds(i * 16, 16)]
            cs = plsc.cumsum(h) + run
            lt = cs < kz_lo
            nl = nl + jnp.where(lt, 1.0, 0.0)
            cl = cl + jnp.where(lt, h, 0.0)
            lt = cs < kz_hi
            nh = nh + jnp.where(lt, 1.0, 0.0)
            ch = ch + jnp.where(lt, h, 0.0)
            run = run + _splat(jnp.sum(h))
            return nl, cl, nh, ch, run

        nl, cl, nh, ch, _ = lax.fori_loop(
            0, HB // 16, pick2_body,
            (zeros16, zeros16, zeros16, zeros16, zeros16),
        )
        b0_lo = _splat(jnp.sum(nl)).astype(jnp.int32)
        k_lo = k_lo0 - _splat(jnp.sum(cl)) - jnp.where(b0_lo > 0, z, zeros16)
        b0_hi = _splat(jnp.sum(nh)).astype(jnp.int32)
        k_hi = k_hi0 - _splat(jnp.sum(ch)) - jnp.where(b0_hi > 0, z, zeros16)

        neq = b0_lo != b0_hi

        def compact_both(i, acc):
            w_lo, w_hi, sb_lo, sb_hi = acc
            for u in range(UNROLL):
                v = col_v[pl.ds(i * STEP + u * 16, 16)]
                nz = v > 0.0
                bits = lax.bitcast_convert_type(v, jnp.int32)
                f0 = lax.shift_right_logical(bits, 21)
                sb_lo = sb_lo + jnp.where(f0 < b0_lo, v, 0.0)
                sb_hi = sb_hi + jnp.where(f0 < b0_hi, v, 0.0)
                m_lo = (f0 == b0_lo) & nz
                c_lo = plsc.cumsum(m_lo.astype(jnp.int32))
                idx_lo = jnp.maximum(w_lo + c_lo - 1, izeros16)
                plsc.store_scatter(cbuf, [idx_lo], v, mask=m_lo)
                w_lo = w_lo + plsc.all_reduce_population_count(m_lo)
                m_hi = (f0 == b0_hi) & nz & neq
                c_hi = plsc.cumsum(m_hi.astype(jnp.int32))
                idx_hi = jnp.clip(CBUF - (w_hi + c_hi), 0, CBUF - 1)
                plsc.store_scatter(cbuf, [idx_hi], v, mask=m_hi)
                w_hi = w_hi + plsc.all_reduce_population_count(m_hi)
            return w_lo, w_hi, sb_lo, sb_hi

        w_lo, w_hi, sbv_lo, sbv_hi = lax.fori_loop(
            0, FULL_ITERS, compact_both, (izeros16, izeros16, zeros16, zeros16)
        )
        s_below_lo = _splat(jnp.sum(sbv_lo))
        s_below_hi = _splat(jnp.sum(sbv_hi))

        if j < COLS_PER_W - 1:
            dma = pltpu.async_copy(ht_hbm.at[col0 + j + 1], col_v, dma_sem)
        eq_s = jnp.max(b0_lo) == jnp.max(b0_hi)
        start_lo = 0
        start_hi = jnp.where(eq_s, 0, CBUF - jnp.max(w_hi))
        nk_lo = w_lo
        nk_hi = jnp.where(neq, w_hi, w_lo)

        def refine(k_rem, b0, start_s, nk):
            zb = jnp.where(b0 == 0, z, zeros16)
            pfx = b0
            nv = nvecs(nk)
            picked = []
            for sh in (14, 7, 0):
                zero_hist(RB // 16)

                def hl(i, c, _sh=sh, _picked=tuple(picked)):
                    v = cbuf[pl.ds(start_s + i * 16, 16)]
                    m = (i * 16 + lane) < nk
                    bits = lax.bitcast_convert_type(v, jnp.int32)
                    for psh, pb in _picked:
                        fp = jnp.bitwise_and(
                            lax.shift_right_logical(bits, psh), 127
                        )
                        m = m & (fp == pb)
                    fl = jnp.bitwise_and(lax.shift_right_logical(bits, _sh), 127)
                    plsc.addupdate_scatter(hist, [fl], ones, mask=m)
                    return c

                lax.fori_loop(0, nv, hl, 0)
                add_zeros_to_bucket0(zb)
                bl, cal = pick(RB // 16, k_rem)
                k_rem = k_rem - cal
                zb = jnp.where(bl == 0, zb, zeros16)
                pfx = lax.shift_left(pfx, 7) + bl
                picked.append((sh, bl))
            return lax.bitcast_convert_type(pfx, jnp.float32), k_rem

        t_lo, krem_lo = refine(k_lo, b0_lo, start_lo, nk_lo)
        t_hi, krem_hi = refine(k_hi, b0_hi, start_hi, nk_hi)
        c_lt_lo = k_lo0 - krem_lo
        c_lt_hi = k_hi0 - krem_hi

        s_lt_lo = s_below_lo + region_sum_lt(start_lo, nk_lo, t_lo)
        s_lt_hi = s_below_hi + region_sum_lt(start_hi, nk_hi, t_hi)

        bot = s_lt_lo + (f_v - c_lt_lo) * t_lo
        top_rm = (total - s_lt_hi) - (n_v - c_lt_hi - f_v) * t_hi
        hbar = (total - bot - top_rm) * (1.0 / (N_ROWS - 2 * F_TRIM))
        res = jnp.where(lane == j, hbar, res)

    res_v[...] = res
    pltpu.sync_copy(res_v, out_hbm.at[wid])


def _dec_kernel(h_ref, w2_ref, b2_ref, o_ref):
    o_ref[...] = (
        jnp.dot(h_ref[...], w2_ref[...], preferred_element_type=jnp.float32)
        + b2_ref[...]
    )


def _decode(hbar, W2p, b2p):
    return pl.pallas_call(
        _dec_kernel,
        out_shape=jax.ShapeDtypeStruct((1, 128), jnp.float32),
    )(hbar, W2p, b2p)


def kernel(x, W1, b1, W2, b2):
    xp = jnp.zeros((N_PAD, D_IN), jnp.float32).at[:N_ROWS].set(x)
    ht = _matmul_T(xp, W1, b1.reshape(HID, 1))
    sel = _sc_select(ht)
    hbar = sel[:, :COLS_PER_W].reshape(1, HID)
    W2p = jnp.zeros((HID, 128), jnp.float32).at[:, :C_OUT].set(W2)
    b2p = jnp.zeros((1, 128), jnp.float32).at[0, :C_OUT].set(b2)
    return _decode(hbar, W2p, b2p)[0, :C_OUT]
